# baseline (speedup 1.0000x reference)
import functools

import jax
import jax.numpy as jnp
from jax import lax
from jax.experimental import pallas as pl
from jax.experimental.pallas import tpu as pltpu

K = 100000
Q = 16
F = 64
T = 20
P = 32
NC = 4
FC = F // NC


def _body(xT_ref, dummy_ref, audioT_hbm, poseT_hbm, outT_ref, loss_ref,
          ablk_ref, idxv_ref, idxs_ref, tiles_ref, asem, gsem, isem):
    for c in range(NC):
        pltpu.make_async_copy(
            audioT_hbm.at[pl.ds(c * FC, FC), :],
            ablk_ref.at[pl.ds(c * FC, FC), :],
            asem.at[c],
        ).start()

    xm = jnp.mean(xT_ref[...], axis=0)

    dotT = None
    rn = None
    for c in range(NC):
        pltpu.make_async_copy(
            audioT_hbm.at[pl.ds(c * FC, FC), :],
            ablk_ref.at[pl.ds(c * FC, FC), :],
            asem.at[c],
        ).wait()
        blkc = ablk_ref[c * FC:(c + 1) * FC, :]
        d = lax.dot_general(xm[:, c * FC:(c + 1) * FC], blkc,
                            (((1,), (0,)), ((), ())),
                            preferred_element_type=jnp.float32,
                            precision=lax.Precision.HIGHEST)
        dotT = d if dotT is None else dotT + d
        for s in range(FC // 8):
            ch = ablk_ref[c * FC + 8 * s:c * FC + 8 * (s + 1), :]
            r = jnp.sum(ch * ch, axis=0, keepdims=True)
            rn = r if rn is None else rn + r

    scores = rn - 2.0 * dotT
    bmin = jnp.min(scores, axis=1, keepdims=True)
    cols = lax.broadcasted_iota(jnp.int32, (Q, K), 1)
    bidx = jnp.min(jnp.where(scores == bmin, cols, K),
                   axis=1, keepdims=True)
    idxv_ref[...] = bidx
    loss_ref[...] = jnp.sum(dummy_ref[...], keepdims=True)

    pltpu.make_async_copy(idxv_ref, idxs_ref, isem).start()
    pltpu.make_async_copy(idxv_ref, idxs_ref, isem).wait()

    def tile_copy(q):
        base = pl.multiple_of((idxs_ref[q, 0] // 128) * 128, 128)
        return pltpu.make_async_copy(
            poseT_hbm.at[:, :, pl.ds(base, 128)],
            tiles_ref.at[pl.ds(q * T, T)],
            gsem,
        )

    for q in range(Q):
        tile_copy(q).start()
    for q in range(Q):
        tile_copy(q).wait()
        lane = idxs_ref[q, 0] % 128
        tile = tiles_ref[q * T:(q + 1) * T]
        rolled = pltpu.roll(tile, (128 - lane) % 128, 2)
        outT_ref[:, :, q:q + 1] = rolled[:, :, 0:1]


_fused_call = pl.pallas_call(
    _body,
    in_specs=[
        pl.BlockSpec((T, Q, F), lambda: (0, 0, 0)),
        pl.BlockSpec((1, 1), lambda: (0, 0)),
        pl.BlockSpec(memory_space=pltpu.MemorySpace.HBM),
        pl.BlockSpec(memory_space=pltpu.MemorySpace.HBM),
    ],
    out_specs=[
        pl.BlockSpec((T, P, Q), lambda: (0, 0, 0)),
        pl.BlockSpec((1, 1), lambda: (0, 0)),
    ],
    out_shape=[
        jax.ShapeDtypeStruct((T, P, Q), jnp.float32),
        jax.ShapeDtypeStruct((1, 1), jnp.float32),
    ],
    scratch_shapes=[
        pltpu.VMEM((F, K), jnp.float32),
        pltpu.VMEM((Q, 1), jnp.int32),
        pltpu.SMEM((Q, 1), jnp.int32),
        pltpu.VMEM((Q * T, P, 128), jnp.float32),
        pltpu.SemaphoreType.DMA((NC,)),
        pltpu.SemaphoreType.DMA,
        pltpu.SemaphoreType.DMA,
    ],
)


@jax.jit
def kernel(x, y, audio, pose, dummy):
    xT = lax.transpose(x[0], (1, 0, 2))
    audioT = lax.transpose(audio, (1, 0))
    poseT = lax.transpose(pose, (1, 2, 0))
    outT, loss = _fused_call(xT, dummy.reshape(1, 1), audioT, poseT)
    out = lax.transpose(outT, (2, 0, 1))
    return (out, loss[0, 0])


# speedup vs baseline: 1.3558x; 1.3558x over previous
import functools

import jax
import jax.numpy as jnp
from jax import lax
from jax.experimental import pallas as pl
from jax.experimental.pallas import tpu as pltpu

K = 100000
Q = 16
F = 64
T = 20
P = 32
NC = 1
FC = F // NC


def _body(xT_ref, dummy_ref, audioT_hbm, poseT_hbm, outT_ref, loss_ref,
          ablk_ref, idxv_ref, idxs_ref, tiles_ref, asem, gsem, isem):
    for c in range(NC):
        pltpu.make_async_copy(
            audioT_hbm.at[pl.ds(c * FC, FC), :],
            ablk_ref.at[pl.ds(c * FC, FC), :],
            asem.at[c],
        ).start()

    xm = jnp.mean(xT_ref[...], axis=0)

    dotT = None
    rn = None
    for c in range(NC):
        pltpu.make_async_copy(
            audioT_hbm.at[pl.ds(c * FC, FC), :],
            ablk_ref.at[pl.ds(c * FC, FC), :],
            asem.at[c],
        ).wait()
        blkc = ablk_ref[c * FC:(c + 1) * FC, :]
        d = lax.dot_general(xm[:, c * FC:(c + 1) * FC], blkc,
                            (((1,), (0,)), ((), ())),
                            preferred_element_type=jnp.float32,
                            precision=lax.Precision.HIGHEST)
        dotT = d if dotT is None else dotT + d
        for s in range(FC // 8):
            ch = ablk_ref[c * FC + 8 * s:c * FC + 8 * (s + 1), :]
            r = jnp.sum(ch * ch, axis=0, keepdims=True)
            rn = r if rn is None else rn + r

    scores = rn - 2.0 * dotT
    bmin = jnp.min(scores, axis=1, keepdims=True)
    cols = lax.broadcasted_iota(jnp.int32, (Q, K), 1)
    bidx = jnp.min(jnp.where(scores == bmin, cols, K),
                   axis=1, keepdims=True)
    idxv_ref[...] = bidx
    loss_ref[...] = jnp.sum(dummy_ref[...], keepdims=True)

    pltpu.make_async_copy(idxv_ref, idxs_ref, isem).start()
    pltpu.make_async_copy(idxv_ref, idxs_ref, isem).wait()

    def tile_copy(q):
        base = pl.multiple_of((idxs_ref[q, 0] // 128) * 128, 128)
        return pltpu.make_async_copy(
            poseT_hbm.at[:, :, pl.ds(base, 128)],
            tiles_ref.at[pl.ds(q * T, T)],
            gsem,
        )

    for q in range(Q):
        tile_copy(q).start()
    for q in range(Q):
        tile_copy(q).wait()
        lane = idxs_ref[q, 0] % 128
        tile = tiles_ref[q * T:(q + 1) * T]
        rolled = pltpu.roll(tile, (128 - lane) % 128, 2)
        outT_ref[:, :, q:q + 1] = rolled[:, :, 0:1]


_fused_call = pl.pallas_call(
    _body,
    in_specs=[
        pl.BlockSpec((T, Q, F), lambda: (0, 0, 0)),
        pl.BlockSpec((1, 1), lambda: (0, 0)),
        pl.BlockSpec(memory_space=pltpu.MemorySpace.HBM),
        pl.BlockSpec(memory_space=pltpu.MemorySpace.HBM),
    ],
    out_specs=[
        pl.BlockSpec((T, P, Q), lambda: (0, 0, 0)),
        pl.BlockSpec((1, 1), lambda: (0, 0)),
    ],
    out_shape=[
        jax.ShapeDtypeStruct((T, P, Q), jnp.float32),
        jax.ShapeDtypeStruct((1, 1), jnp.float32),
    ],
    scratch_shapes=[
        pltpu.VMEM((F, K), jnp.float32),
        pltpu.VMEM((Q, 1), jnp.int32),
        pltpu.SMEM((Q, 1), jnp.int32),
        pltpu.VMEM((Q * T, P, 128), jnp.float32),
        pltpu.SemaphoreType.DMA((NC,)),
        pltpu.SemaphoreType.DMA,
        pltpu.SemaphoreType.DMA,
    ],
)


@jax.jit
def kernel(x, y, audio, pose, dummy):
    xT = lax.transpose(x[0], (1, 0, 2))
    audioT = lax.transpose(audio, (1, 0))
    poseT = lax.transpose(pose, (1, 2, 0))
    outT, loss = _fused_call(xT, dummy.reshape(1, 1), audioT, poseT)
    out = lax.transpose(outT, (2, 0, 1))
    return (out, loss[0, 0])


# speedup vs baseline: 1.4673x; 1.0822x over previous
import functools

import jax
import jax.numpy as jnp
from jax import lax
from jax.experimental import pallas as pl
from jax.experimental.pallas import tpu as pltpu

K = 100000
Q = 16
F = 64
T = 20
P = 32
NC = 4
FC = F // NC


def _body(xT_ref, dummy_ref, audioT_hbm, poseT_hbm, outT_ref, loss_ref,
          ablk_ref, idxv_ref, idxs_ref, tiles_ref, asem, gsem, isem):
    for c in range(NC):
        pltpu.make_async_copy(
            audioT_hbm.at[pl.ds(c * FC, FC), :],
            ablk_ref.at[pl.ds(c * FC, FC), :],
            asem.at[c],
        ).start()

    xm = jnp.mean(xT_ref[...], axis=0)

    rn = None
    for c in range(NC):
        pltpu.make_async_copy(
            audioT_hbm.at[pl.ds(c * FC, FC), :],
            ablk_ref.at[pl.ds(c * FC, FC), :],
            asem.at[c],
        ).wait()
        for s in range(FC // 8):
            ch = ablk_ref[c * FC + 8 * s:c * FC + 8 * (s + 1), :]
            r = jnp.sum(ch * ch, axis=0, keepdims=True)
            rn = r if rn is None else rn + r

    dotT = lax.dot_general(xm, ablk_ref[...], (((1,), (0,)), ((), ())),
                           preferred_element_type=jnp.float32,
                           precision=lax.Precision.HIGHEST)
    scores = rn - 2.0 * dotT
    bmin = jnp.min(scores, axis=1, keepdims=True)
    cols = lax.broadcasted_iota(jnp.int32, (Q, K), 1)
    bidx = jnp.min(jnp.where(scores == bmin, cols, K),
                   axis=1, keepdims=True)
    idxv_ref[...] = bidx
    loss_ref[...] = jnp.sum(dummy_ref[...], keepdims=True)

    pltpu.make_async_copy(idxv_ref, idxs_ref, isem).start()
    pltpu.make_async_copy(idxv_ref, idxs_ref, isem).wait()

    def tile_copy(q):
        base = pl.multiple_of((idxs_ref[q, 0] // 128) * 128, 128)
        return pltpu.make_async_copy(
            poseT_hbm.at[:, :, pl.ds(base, 128)],
            tiles_ref.at[pl.ds(q * T, T)],
            gsem,
        )

    for q in range(Q):
        tile_copy(q).start()
    for q in range(Q):
        tile_copy(q).wait()
        lane = idxs_ref[q, 0] % 128
        tile = tiles_ref[q * T:(q + 1) * T]
        rolled = pltpu.roll(tile, (128 - lane) % 128, 2)
        outT_ref[:, :, q:q + 1] = rolled[:, :, 0:1]


_fused_call = pl.pallas_call(
    _body,
    in_specs=[
        pl.BlockSpec((T, Q, F), lambda: (0, 0, 0)),
        pl.BlockSpec((1, 1), lambda: (0, 0)),
        pl.BlockSpec(memory_space=pltpu.MemorySpace.HBM),
        pl.BlockSpec(memory_space=pltpu.MemorySpace.HBM),
    ],
    out_specs=[
        pl.BlockSpec((T, P, Q), lambda: (0, 0, 0)),
        pl.BlockSpec((1, 1), lambda: (0, 0)),
    ],
    out_shape=[
        jax.ShapeDtypeStruct((T, P, Q), jnp.float32),
        jax.ShapeDtypeStruct((1, 1), jnp.float32),
    ],
    scratch_shapes=[
        pltpu.VMEM((F, K), jnp.float32),
        pltpu.VMEM((Q, 1), jnp.int32),
        pltpu.SMEM((Q, 1), jnp.int32),
        pltpu.VMEM((Q * T, P, 128), jnp.float32),
        pltpu.SemaphoreType.DMA((NC,)),
        pltpu.SemaphoreType.DMA,
        pltpu.SemaphoreType.DMA,
    ],
)


@jax.jit
def kernel(x, y, audio, pose, dummy):
    xT = lax.transpose(x[0], (1, 0, 2))
    audioT = lax.transpose(audio, (1, 0))
    poseT = lax.transpose(pose, (1, 2, 0))
    outT, loss = _fused_call(xT, dummy.reshape(1, 1), audioT, poseT)
    out = lax.transpose(outT, (2, 0, 1))
    return (out, loss[0, 0])
